# baseline (speedup 1.0000x reference)
import jax
import jax.numpy as jnp
from jax import lax
from jax.experimental import pallas as pl
from jax.experimental.pallas import tpu as pltpu

E = 16
D = 2048
DE = 128
B = 4
P = 2048
S = D // E
H = S // 4
PBLK = 1024
NBLK = P // PBLK


def _body(tok_ref, fp_ref, gW1r_ref, gb1r_ref, gW2f_ref, gb2r_ref,
          alpha_ref, wqkv_ref, penta_ref, fw_ref, temp_ref, out_ref,
          ka_s, qa_s, v_s):
    pb = pl.program_id(1)

    @pl.when(pb < NBLK)
    def _stage1():
        tok = tok_ref[0]
        fp = fp_ref[...]
        a = jnp.clip(jnp.floor(fp * E).astype(jnp.int32), 0, E - 1)
        eidx = lax.broadcasted_iota(jnp.int32, (PBLK, E), 1)
        onehot = (eidx == a).astype(jnp.float32)
        xs = tok[:, 0:S] * onehot[:, 0:1]
        for e in range(1, E):
            xs = xs + tok[:, e * S:(e + 1) * S] * onehot[:, e:e + 1]
        t = jnp.dot(xs, gW1r_ref[...]) + gb1r_ref[...]
        gel = jax.nn.gelu(t)
        gpre_all = jnp.dot(gel, gW2f_ref[...]) + gb2r_ref[...]
        gsel = jnp.sum(gpre_all * onehot, axis=-1, keepdims=True)
        g = jax.nn.sigmoid(gsel)
        aw = jnp.dot(onehot, jax.nn.sigmoid(alpha_ref[...]))
        feat = xs * (g * aw + (1.0 - aw))
        u = jnp.concatenate(
            [feat * onehot[:, e:e + 1] for e in range(E)], axis=1)
        qkv = jnp.dot(u, wqkv_ref[...])
        q = qkv[:, 0:DE]
        k = qkv[:, DE:2 * DE]
        v_s[pl.ds(pb * PBLK, PBLK), :] = qkv[:, 2 * DE:3 * DE]
        dn = []
        for vtx in range(5):
            pv = penta_ref[:, vtx * DE:(vtx + 1) * DE]
            nrm = jnp.sqrt(jnp.sum(pv * pv, axis=-1, keepdims=True))
            dn.append(pv / (nrm + 1e-8))
        dall = jnp.dot(onehot, jnp.concatenate(dn, axis=1))
        kas, qas = [], []
        for vtx in range(5):
            dv = dall[:, vtx * DE:(vtx + 1) * DE]
            kas.append(jnp.sum(k * dv, axis=-1, keepdims=True))
            qas.append(jnp.sum(q * dv, axis=-1, keepdims=True))
        ka_s[pl.ds(pb * PBLK, PBLK), :] = jnp.concatenate(kas, axis=1)
        qa_s[pl.ds(pb * PBLK, PBLK), :] = jnp.concatenate(qas, axis=1)

    @pl.when(pb == NBLK)
    def _stage2():
        ka = ka_s[...] / temp_ref[0, 0]
        mx = jnp.max(ka, axis=0, keepdims=True)
        ex = jnp.exp(ka - mx)
        w = ex / jnp.sum(ex, axis=0, keepdims=True)
        ctx = lax.dot_general(w, v_s[...], (((0,), (0,)), ((), ())))
        qf = qa_s[...] * fw_ref[...]
        out_ref[0] = jnp.dot(qf, ctx)


@jax.jit
def kernel(tokens, fingerprints, Wq, Wk, Wv, alpha, gW1, gb1, gW2, gb2,
           penta, fusion_w, temperature):
    gW1r = gW1.transpose(1, 0, 2).reshape(S, E * H)
    gb1r = gb1.reshape(1, E * H)
    gW2f = (gW2[:, :, 0][:, :, None] * jnp.eye(E)[:, None, :]).reshape(E * H, E)
    gb2r = gb2.reshape(1, E)
    wqkv = jnp.concatenate(
        [Wq.reshape(E * S, DE), Wk.reshape(E * S, DE), Wv.reshape(E * S, DE)],
        axis=1)
    alpha2 = alpha.reshape(E, 1)
    penta640 = penta.reshape(E, 5 * DE)
    fp2 = fingerprints.reshape(P, 1)
    fw2 = fusion_w.reshape(1, 5)
    temp2 = temperature.reshape(1, 1)

    full = lambda b, pb: (0, 0)
    clamp = lambda pb: jnp.minimum(pb, NBLK - 1)
    out = pl.pallas_call(
        _body,
        grid=(B, NBLK + 1),
        in_specs=[
            pl.BlockSpec((1, PBLK, D), lambda b, pb: (b, clamp(pb), 0)),
            pl.BlockSpec((PBLK, 1), lambda b, pb: (clamp(pb), 0)),
            pl.BlockSpec((S, E * H), full),
            pl.BlockSpec((1, E * H), full),
            pl.BlockSpec((E * H, E), full),
            pl.BlockSpec((1, E), full),
            pl.BlockSpec((E, 1), full),
            pl.BlockSpec((E * S, 3 * DE), full),
            pl.BlockSpec((E, 5 * DE), full),
            pl.BlockSpec((1, 5), full),
            pl.BlockSpec((1, 1), full),
        ],
        out_specs=pl.BlockSpec((1, P, DE), lambda b, pb: (b, 0, 0)),
        out_shape=jax.ShapeDtypeStruct((B, P, DE), jnp.float32),
        scratch_shapes=[
            pltpu.VMEM((P, 5), jnp.float32),
            pltpu.VMEM((P, 5), jnp.float32),
            pltpu.VMEM((P, DE), jnp.float32),
        ],
        compiler_params=pltpu.CompilerParams(
            dimension_semantics=("arbitrary", "arbitrary")),
    )(tokens, fp2, gW1r, gb1r, gW2f, gb2r, alpha2, wqkv, penta640, fw2, temp2)
    return out


# speedup vs baseline: 1.0995x; 1.0995x over previous
import jax
import jax.numpy as jnp
from jax import lax
from jax.experimental import pallas as pl
from jax.experimental.pallas import tpu as pltpu

E = 16
D = 2048
DE = 128
B = 4
P = 2048
S = D // E
H = S // 4
PBLK = 1024


def _stage1_body(tok_ref, fp_ref, gW1r_ref, gb1r_ref, gW2f_ref, gb2r_ref,
                 alpha_ref, wqkv_ref, penta_ref, qa_ref, ka_ref, v_ref):
    tok = tok_ref[0]
    fp = fp_ref[...]
    a = jnp.clip(jnp.floor(fp * E).astype(jnp.int32), 0, E - 1)
    eidx = lax.broadcasted_iota(jnp.int32, (PBLK, E), 1)
    onehot = (eidx == a).astype(jnp.float32)
    xs = tok[:, 0:S] * onehot[:, 0:1]
    for e in range(1, E):
        xs = xs + tok[:, e * S:(e + 1) * S] * onehot[:, e:e + 1]
    t = jnp.dot(xs, gW1r_ref[...]) + gb1r_ref[...]
    gel = jax.nn.gelu(t)
    gpre_all = jnp.dot(gel, gW2f_ref[...]) + gb2r_ref[...]
    gsel = jnp.sum(gpre_all * onehot, axis=-1, keepdims=True)
    g = jax.nn.sigmoid(gsel)
    aw = jnp.dot(onehot, jax.nn.sigmoid(alpha_ref[...]))
    feat = xs * (g * aw + (1.0 - aw))
    u = jnp.concatenate([feat * onehot[:, e:e + 1] for e in range(E)], axis=1)
    qkv = jnp.dot(u, wqkv_ref[...])
    q = qkv[:, 0:DE]
    k = qkv[:, DE:2 * DE]
    v_ref[0] = qkv[:, 2 * DE:3 * DE]
    dn = []
    for vtx in range(5):
        pv = penta_ref[:, vtx * DE:(vtx + 1) * DE]
        nrm = jnp.sqrt(jnp.sum(pv * pv, axis=-1, keepdims=True))
        dn.append(pv / (nrm + 1e-8))
    dall = jnp.dot(onehot, jnp.concatenate(dn, axis=1))
    kas, qas = [], []
    for vtx in range(5):
        dv = dall[:, vtx * DE:(vtx + 1) * DE]
        kas.append(jnp.sum(k * dv, axis=-1, keepdims=True))
        qas.append(jnp.sum(q * dv, axis=-1, keepdims=True))
    ka_ref[0] = jnp.concatenate(kas, axis=1)
    qa_ref[0] = jnp.concatenate(qas, axis=1)


def _stage2_body(ka_ref, qa_ref, v_ref, fw_ref, temp_ref, out_ref):
    ka = ka_ref[0] / temp_ref[0, 0]
    mx = jnp.max(ka, axis=0, keepdims=True)
    ex = jnp.exp(ka - mx)
    w = ex / jnp.sum(ex, axis=0, keepdims=True)
    ctx = lax.dot_general(w, v_ref[0], (((0,), (0,)), ((), ())))
    qf = qa_ref[0] * fw_ref[...]
    out_ref[0] = jnp.dot(qf, ctx)


@jax.jit
def kernel(tokens, fingerprints, Wq, Wk, Wv, alpha, gW1, gb1, gW2, gb2,
           penta, fusion_w, temperature):
    gW1r = gW1.transpose(1, 0, 2).reshape(S, E * H)
    gb1r = gb1.reshape(1, E * H)
    gW2f = (gW2[:, :, 0][:, :, None] * jnp.eye(E)[:, None, :]).reshape(E * H, E)
    gb2r = gb2.reshape(1, E)
    wqkv = jnp.concatenate(
        [Wq.reshape(E * S, DE), Wk.reshape(E * S, DE), Wv.reshape(E * S, DE)],
        axis=1)
    alpha2 = alpha.reshape(E, 1)
    penta640 = penta.reshape(E, 5 * DE)
    fp2 = fingerprints.reshape(P, 1)
    fw2 = fusion_w.reshape(1, 5)
    temp2 = temperature.reshape(1, 1)

    nblk = P // PBLK
    full = lambda i, j: (0, 0)
    qa, ka, v = pl.pallas_call(
        _stage1_body,
        grid=(B, nblk),
        in_specs=[
            pl.BlockSpec((1, PBLK, D), lambda b, pb: (b, pb, 0)),
            pl.BlockSpec((PBLK, 1), lambda b, pb: (pb, 0)),
            pl.BlockSpec((S, E * H), full),
            pl.BlockSpec((1, E * H), full),
            pl.BlockSpec((E * H, E), full),
            pl.BlockSpec((1, E), full),
            pl.BlockSpec((E, 1), full),
            pl.BlockSpec((E * S, 3 * DE), full),
            pl.BlockSpec((E, 5 * DE), full),
        ],
        out_specs=[
            pl.BlockSpec((1, PBLK, 5), lambda b, pb: (b, pb, 0)),
            pl.BlockSpec((1, PBLK, 5), lambda b, pb: (b, pb, 0)),
            pl.BlockSpec((1, PBLK, DE), lambda b, pb: (b, pb, 0)),
        ],
        out_shape=[
            jax.ShapeDtypeStruct((B, P, 5), jnp.float32),
            jax.ShapeDtypeStruct((B, P, 5), jnp.float32),
            jax.ShapeDtypeStruct((B, P, DE), jnp.float32),
        ],
        compiler_params=pltpu.CompilerParams(
            dimension_semantics=("parallel", "parallel")),
    )(tokens, fp2, gW1r, gb1r, gW2f, gb2r, alpha2, wqkv, penta640)

    out = pl.pallas_call(
        _stage2_body,
        grid=(B,),
        in_specs=[
            pl.BlockSpec((1, P, 5), lambda b: (b, 0, 0)),
            pl.BlockSpec((1, P, 5), lambda b: (b, 0, 0)),
            pl.BlockSpec((1, P, DE), lambda b: (b, 0, 0)),
            pl.BlockSpec((1, 5), lambda b: (0, 0)),
            pl.BlockSpec((1, 1), lambda b: (0, 0)),
        ],
        out_specs=pl.BlockSpec((1, P, DE), lambda b: (b, 0, 0)),
        out_shape=jax.ShapeDtypeStruct((B, P, DE), jnp.float32),
        compiler_params=pltpu.CompilerParams(
            dimension_semantics=("parallel",)),
    )(ka, qa, v, fw2, temp2)
    return out
